# Initial kernel scaffold; baseline (speedup 1.0000x reference)
import jax
import jax.numpy as jnp
from jax.experimental import pallas as pl


def kernel(x_nchw, text, t_w, t_b, e1_w, e1_scale, e1_shift, e2_wx, e2_wt, e2_scale, e2_shift, e3_wx, e3_wt, e3_scale, e3_shift, d1_w, d1_scale, d1_shift, d2_wx, d2_wt, d2_scale, d2_shift, d3a_wx, d3a_wt, d3a_scale, d3a_shift, d3b_w, d3b_scale, d3b_shift):
    raise NotImplementedError("write your pallas kernel here")


# speedup vs baseline: 1.2822x; 1.2822x over previous
import functools

import jax
import jax.numpy as jnp
from jax.experimental import pallas as pl
from jax.experimental.pallas import tpu as pltpu

f32 = jnp.float32


def _conv_body(*refs, TH, W, Wp, Cin, Cout, H, act, n_tiles, pool, has_tb):
    if has_tb:
        xm, xtop, xbot, w_ref, sc_ref, sh_ref, tb_ref, o_ref, xbuf = refs
    else:
        xm, xtop, xbot, w_ref, sc_ref, sh_ref, o_ref, xbuf = refs
        tb_ref = None
    i = pl.program_id(1)
    nf = (i > 0).astype(f32)
    nl = (i < n_tiles - 1).astype(f32)
    xbuf[:, W:Wp, :] = jnp.zeros((TH + 4, Wp - W, Cin), f32)
    xbuf[0:1, 0:W, :] = jnp.zeros((1, W, Cin), f32)
    xbuf[TH + 3:TH + 4, 0:W, :] = jnp.zeros((1, W, Cin), f32)
    xbuf[2:TH + 2, 0:W, :] = xm[0]
    xbuf[1:2, 0:W, :] = xtop[0] * nf
    xbuf[TH + 2:TH + 3, 0:W, :] = xbot[0] * nl

    flat = xbuf[...].reshape((TH + 4) * Wp, Cin)
    acc = jnp.zeros((TH * Wp, Cout), f32)
    for dw in (-1, 0, 1):
        xs = flat[Wp + dw: Wp + dw + (TH + 2) * Wp, :]
        for kh in range(3):
            acc = acc + jnp.dot(xs[kh * Wp:(kh + TH) * Wp, :],
                                w_ref[kh * 3 + dw + 1],
                                preferred_element_type=f32)
    acc = acc * sc_ref[...] + sh_ref[...]
    a3 = acc.reshape(TH, Wp, Cout)

    if tb_ref is not None:
        tb = tb_ref[0]

        def v(k):
            return tb[k:k + 1, :].reshape(1, 1, Cout)

        rr = jax.lax.broadcasted_iota(jnp.int32, (TH, 1, 1), 0) + i * TH
        mt = (rr == 0).astype(f32)
        mb = (rr == H - 1).astype(f32)
        cc = jax.lax.broadcasted_iota(jnp.int32, (1, Wp, 1), 1)
        ml = (cc == 0).astype(f32)
        mr = (cc == W - 1).astype(f32)
        a3 = a3 + (v(0) + mt * v(1) + mb * v(2))
        a3 = a3 + ml * (v(3) + mt * v(5) + mb * v(7))
        a3 = a3 + mr * (v(4) + mt * v(6) + mb * v(8))

    if act == "relu":
        a3 = jnp.maximum(a3, 0.0)
    elif act == "sigmoid":
        a3 = 1.0 / (1.0 + jnp.exp(-a3))

    if pool:
        a = a3[:, 0:W, :].reshape(TH // 2, 2, W, Cout).max(axis=1)
        a = a.reshape(TH // 2, W // 2, 2, Cout).max(axis=2)
        o_ref[0] = a
    else:
        o_ref[0] = a3[:, 0:W, :]


def _conv(x, w9, scale, shift, *, act, tb=None, pool=False, TH=32):
    B, H, W, Cin = x.shape
    Cout = w9.shape[-1]
    TH = min(TH, H)
    n_tiles = H // TH
    Wp = W + 8
    body = functools.partial(
        _conv_body, TH=TH, W=W, Wp=Wp, Cin=Cin, Cout=Cout, H=H, act=act,
        n_tiles=n_tiles, pool=pool, has_tb=tb is not None)
    in_specs = [
        pl.BlockSpec((1, TH, W, Cin), lambda b, i: (b, i, 0, 0)),
        pl.BlockSpec((1, 1, W, Cin),
                     lambda b, i: (b, jnp.maximum(i * TH - 1, 0), 0, 0)),
        pl.BlockSpec((1, 1, W, Cin),
                     lambda b, i: (b, jnp.minimum(i * TH + TH, H - 1), 0, 0)),
        pl.BlockSpec((9, Cin, Cout), lambda b, i: (0, 0, 0)),
        pl.BlockSpec((1, Cout), lambda b, i: (0, 0)),
        pl.BlockSpec((1, Cout), lambda b, i: (0, 0)),
    ]
    inputs = [x, x, x, w9, scale.reshape(1, Cout), shift.reshape(1, Cout)]
    if tb is not None:
        in_specs.append(pl.BlockSpec((1, 9, Cout), lambda b, i: (b, 0, 0)))
        inputs.append(tb)
    if pool:
        out_shape = jax.ShapeDtypeStruct((B, H // 2, W // 2, Cout), f32)
        out_spec = pl.BlockSpec((1, TH // 2, W // 2, Cout),
                                lambda b, i: (b, i, 0, 0))
    else:
        out_shape = jax.ShapeDtypeStruct((B, H, W, Cout), f32)
        out_spec = pl.BlockSpec((1, TH, W, Cout), lambda b, i: (b, i, 0, 0))
    return pl.pallas_call(
        body,
        out_shape=out_shape,
        grid_spec=pltpu.PrefetchScalarGridSpec(
            num_scalar_prefetch=0,
            grid=(B, n_tiles),
            in_specs=in_specs,
            out_specs=out_spec,
            scratch_shapes=[pltpu.VMEM((TH + 4, Wp, Cin), f32)]),
        compiler_params=pltpu.CompilerParams(
            dimension_semantics=("parallel", "parallel")),
    )(*inputs)


def _lin_body(x_ref, w_ref, b_ref, o_ref):
    o_ref[...] = (jnp.dot(x_ref[...], w_ref[...],
                          preferred_element_type=f32) + b_ref[...])


def _linear(x, w, b):
    B, K = x.shape
    N = w.shape[-1]
    return pl.pallas_call(
        _lin_body,
        out_shape=jax.ShapeDtypeStruct((B, N), f32),
        grid=(1,),
        in_specs=[pl.BlockSpec((B, K), lambda i: (0, 0)),
                  pl.BlockSpec((K, N), lambda i: (0, 0)),
                  pl.BlockSpec((1, N), lambda i: (0, 0))],
        out_specs=pl.BlockSpec((B, N), lambda i: (0, 0)),
    )(x, w, b.reshape(1, N))


def _wstar9(wt):
    full = wt.sum((0, 1))
    top = -wt[0].sum(0)
    bot = -wt[2].sum(0)
    left = -wt[:, 0].sum(0)
    right = -wt[:, 2].sum(0)
    return jnp.stack([full, top, bot, left, right,
                      wt[0, 0], wt[0, 2], wt[2, 0], wt[2, 2]])


def _bilin_mat(n_in, n_out):
    i = jnp.arange(n_out, dtype=f32)
    src = i * (n_in - 1) / (n_out - 1)
    i0 = jnp.clip(jnp.floor(src).astype(jnp.int32), 0, n_in - 2)
    frac = src - i0.astype(f32)
    rows = jnp.arange(n_out)
    M = jnp.zeros((n_out, n_in), f32)
    M = M.at[rows, i0].add(1.0 - frac)
    M = M.at[rows, i0 + 1].add(frac)
    return M


def _up2(x):
    B, H, W, C = x.shape
    Mh = _bilin_mat(H, 2 * H)
    Mw = _bilin_mat(W, 2 * W)
    x = jnp.einsum('ph,bhwc->bpwc', Mh, x)
    return jnp.einsum('qw,bpwc->bpqc', Mw, x)


def kernel(x_nchw, text, t_w, t_b, e1_w, e1_scale, e1_shift, e2_wx, e2_wt,
           e2_scale, e2_shift, e3_wx, e3_wt, e3_scale, e3_shift, d1_w,
           d1_scale, d1_shift, d2_wx, d2_wt, d2_scale, d2_shift, d3a_wx,
           d3a_wt, d3a_scale, d3a_shift, d3b_w, d3b_scale, d3b_shift):
    B = x_nchw.shape[0]
    x = jnp.transpose(x_nchw, (0, 2, 3, 1)).astype(f32)
    x = jnp.pad(x, ((0, 0), (0, 0), (0, 0), (0, 5)))
    text = text.astype(f32)

    layers = [(e2_wt, e2_scale, 0), (e3_wt, e3_scale, 64),
              (d2_wt, d2_scale, 128), (d3a_wt, d3a_scale, 192)]
    segs, bsegs = [], []
    for wt_, sc_, off in layers:
        ws = _wstar9(wt_) * sc_
        segs.append(jnp.einsum('kc,tco->kto', t_w[:, off:off + 64],
                               ws).reshape(512, -1))
        bsegs.append(jnp.einsum('c,tco->to', t_b[off:off + 64],
                                ws).reshape(-1))
    tball = _linear(text, jnp.concatenate(segs, axis=1),
                    jnp.concatenate(bsegs))
    tbs, o = [], 0
    for wt_, _, _ in layers:
        n = 9 * wt_.shape[-1]
        tbs.append(tball[:, o:o + n].reshape(B, 9, wt_.shape[-1]))
        o += n
    tb_e2, tb_e3, tb_d2, tb_d3a = tbs

    def r9(w):
        return w.reshape(9, w.shape[2], w.shape[3])

    e1w = jnp.pad(e1_w, ((0, 0), (0, 0), (0, 5), (0, 0)))
    h = _conv(x, r9(e1w), e1_scale, e1_shift, act='relu')
    h = _conv(h, r9(e2_wx), e2_scale, e2_shift, act='relu', tb=tb_e2)
    h = _conv(h, r9(e3_wx), e3_scale, e3_shift, act='relu', tb=tb_e3,
              pool=True)
    h = _conv(h, r9(d1_w), d1_scale, d1_shift, act='relu')
    h = _conv(h, r9(d2_wx), d2_scale, d2_shift, act='relu', tb=tb_d2)
    h = _conv(h, r9(d3a_wx), d3a_scale, d3a_shift, act='relu', tb=tb_d3a)
    h = _up2(h)
    h = _conv(h, r9(d3b_w), d3b_scale, d3b_shift, act='sigmoid')
    return jnp.transpose(h, (0, 3, 1, 2))


# speedup vs baseline: 1.5114x; 1.1788x over previous
import functools

import jax
import jax.numpy as jnp
from jax.experimental import pallas as pl
from jax.experimental.pallas import tpu as pltpu

f32 = jnp.float32


def _conv_body(*refs, TH, W, Wp, Cin, Cout, H, act, n_tiles, pool, has_tb,
               hcw=False):
    if has_tb:
        xm, xtop, xbot, w_ref, sc_ref, sh_ref, tb_ref, o_ref, xbuf = refs
    else:
        xm, xtop, xbot, w_ref, sc_ref, sh_ref, o_ref, xbuf = refs
        tb_ref = None
    i = pl.program_id(1)
    nf = (i > 0).astype(f32)
    nl = (i < n_tiles - 1).astype(f32)
    xbuf[:, W:Wp, :] = jnp.zeros((TH + 4, Wp - W, Cin), f32)
    xbuf[0:1, 0:W, :] = jnp.zeros((1, W, Cin), f32)
    xbuf[TH + 3:TH + 4, 0:W, :] = jnp.zeros((1, W, Cin), f32)
    xbuf[2:TH + 2, 0:W, :] = xm[0]
    xbuf[1:2, 0:W, :] = xtop[0] * nf
    xbuf[TH + 2:TH + 3, 0:W, :] = xbot[0] * nl

    flat = xbuf[...].reshape((TH + 4) * Wp, Cin)
    acc = jnp.zeros((TH * Wp, Cout), f32)
    for dw in (-1, 0, 1):
        xs = flat[Wp + dw: Wp + dw + (TH + 2) * Wp, :]
        for kh in range(3):
            acc = acc + jnp.dot(xs[kh * Wp:(kh + TH) * Wp, :],
                                w_ref[kh * 3 + dw + 1],
                                preferred_element_type=f32)
    acc = acc * sc_ref[...] + sh_ref[...]
    a3 = acc.reshape(TH, Wp, Cout)

    if tb_ref is not None:
        tb = tb_ref[0]

        def v(k):
            return tb[k:k + 1, :].reshape(1, 1, Cout)

        rr = jax.lax.broadcasted_iota(jnp.int32, (TH, 1, 1), 0) + i * TH
        mt = (rr == 0).astype(f32)
        mb = (rr == H - 1).astype(f32)
        cc = jax.lax.broadcasted_iota(jnp.int32, (1, Wp, 1), 1)
        ml = (cc == 0).astype(f32)
        mr = (cc == W - 1).astype(f32)
        a3 = a3 + (v(0) + mt * v(1) + mb * v(2))
        a3 = a3 + ml * (v(3) + mt * v(5) + mb * v(7))
        a3 = a3 + mr * (v(4) + mt * v(6) + mb * v(8))

    if act == "relu":
        a3 = jnp.maximum(a3, 0.0)
    elif act == "sigmoid":
        a3 = 1.0 / (1.0 + jnp.exp(-a3))

    if pool:
        a = a3[:, 0:W, :].reshape(TH // 2, 2, W, Cout).max(axis=1)
        a = a.reshape(TH // 2, W // 2, 2, Cout).max(axis=2)
        o_ref[0] = a
    elif hcw:
        o_ref[0] = jnp.swapaxes(a3[:, 0:W, :], 1, 2)
    else:
        o_ref[0] = a3[:, 0:W, :]


def _conv(x, w9, scale, shift, *, act, tb=None, pool=False, hcw=False, TH=32):
    B, H, W, Cin = x.shape
    Cout = w9.shape[-1]
    TH = min(TH, H)
    n_tiles = H // TH
    Wp = W + 8
    body = functools.partial(
        _conv_body, TH=TH, W=W, Wp=Wp, Cin=Cin, Cout=Cout, H=H, act=act,
        n_tiles=n_tiles, pool=pool, has_tb=tb is not None, hcw=hcw)
    in_specs = [
        pl.BlockSpec((1, TH, W, Cin), lambda b, i: (b, i, 0, 0)),
        pl.BlockSpec((1, 1, W, Cin),
                     lambda b, i: (b, jnp.maximum(i * TH - 1, 0), 0, 0)),
        pl.BlockSpec((1, 1, W, Cin),
                     lambda b, i: (b, jnp.minimum(i * TH + TH, H - 1), 0, 0)),
        pl.BlockSpec((9, Cin, Cout), lambda b, i: (0, 0, 0)),
        pl.BlockSpec((1, Cout), lambda b, i: (0, 0)),
        pl.BlockSpec((1, Cout), lambda b, i: (0, 0)),
    ]
    inputs = [x, x, x, w9, scale.reshape(1, Cout), shift.reshape(1, Cout)]
    if tb is not None:
        in_specs.append(pl.BlockSpec((1, 9, Cout), lambda b, i: (b, 0, 0)))
        inputs.append(tb)
    if pool:
        out_shape = jax.ShapeDtypeStruct((B, H // 2, W // 2, Cout), f32)
        out_spec = pl.BlockSpec((1, TH // 2, W // 2, Cout),
                                lambda b, i: (b, i, 0, 0))
    elif hcw:
        out_shape = jax.ShapeDtypeStruct((B, H, Cout, W), f32)
        out_spec = pl.BlockSpec((1, TH, Cout, W), lambda b, i: (b, i, 0, 0))
    else:
        out_shape = jax.ShapeDtypeStruct((B, H, W, Cout), f32)
        out_spec = pl.BlockSpec((1, TH, W, Cout), lambda b, i: (b, i, 0, 0))
    return pl.pallas_call(
        body,
        out_shape=out_shape,
        grid_spec=pltpu.PrefetchScalarGridSpec(
            num_scalar_prefetch=0,
            grid=(B, n_tiles),
            in_specs=in_specs,
            out_specs=out_spec,
            scratch_shapes=[pltpu.VMEM((TH + 4, Wp, Cin), f32)]),
        compiler_params=pltpu.CompilerParams(
            dimension_semantics=("parallel", "parallel")),
    )(*inputs)


def _lin_body(x_ref, w_ref, b_ref, o_ref):
    o_ref[...] = (jnp.dot(x_ref[...], w_ref[...],
                          preferred_element_type=f32) + b_ref[...])


def _linear(x, w, b):
    B, K = x.shape
    N = w.shape[-1]
    return pl.pallas_call(
        _lin_body,
        out_shape=jax.ShapeDtypeStruct((B, N), f32),
        grid=(1,),
        in_specs=[pl.BlockSpec((B, K), lambda i: (0, 0)),
                  pl.BlockSpec((K, N), lambda i: (0, 0)),
                  pl.BlockSpec((1, N), lambda i: (0, 0))],
        out_specs=pl.BlockSpec((B, N), lambda i: (0, 0)),
    )(x, w, b.reshape(1, N))


def _wstar9(wt):
    full = wt.sum((0, 1))
    top = -wt[0].sum(0)
    bot = -wt[2].sum(0)
    left = -wt[:, 0].sum(0)
    right = -wt[:, 2].sum(0)
    return jnp.stack([full, top, bot, left, right,
                      wt[0, 0], wt[0, 2], wt[2, 0], wt[2, 2]])


def _bilin_mat(n_in, n_out):
    i = jnp.arange(n_out, dtype=f32)
    src = i * (n_in - 1) / (n_out - 1)
    i0 = jnp.clip(jnp.floor(src).astype(jnp.int32), 0, n_in - 2)
    frac = src - i0.astype(f32)
    rows = jnp.arange(n_out)
    M = jnp.zeros((n_out, n_in), f32)
    M = M.at[rows, i0].add(1.0 - frac)
    M = M.at[rows, i0 + 1].add(frac)
    return M


def _upconv_body(x_ref, mh_ref, mwt_ref, w9t_ref, sc_ref, sh_ref, o_ref,
                 zbuf, *, TH, Hin, Win, C):
    i = pl.program_id(1)
    xflat = x_ref[0].reshape(Hin, C * Win)
    mh = mh_ref[pl.ds(i * TH, TH + 2), :]
    uph = jnp.dot(mh, xflat, preferred_element_type=f32)
    uph3 = uph.reshape(TH + 2, C, Win)
    w9t = w9t_ref[...]
    for r in range(TH + 2):
        zbuf[r] = jnp.dot(w9t, uph3[r], preferred_element_type=f32)
    zb = zbuf[...]
    acc = jnp.zeros((TH, 2 * Win), f32)
    for kh in range(3):
        for kw in range(3):
            zs = zb[kh:kh + TH, 3 * kh + kw, :]
            acc = acc + jnp.dot(zs, mwt_ref[kw],
                                preferred_element_type=f32)
    acc = acc * sc_ref[0, 0] + sh_ref[0, 0]
    o_ref[0, 0] = 1.0 / (1.0 + jnp.exp(-acc))


def _upconv(x_hcw, w, scale, shift, *, TH=32):
    B, Hin, C, Win = x_hcw.shape
    Hout, Wout = 2 * Hin, 2 * Win
    TH = min(TH, Hout)
    n_tiles = Hout // TH
    Mh = _bilin_mat(Hin, Hout)
    mh_pad = jnp.zeros((Hout + 8, Hin), f32).at[1:Hout + 1, :].set(Mh)
    Mw = _bilin_mat(Win, Wout)
    mw_pad = jnp.zeros((Wout + 2, Win), f32).at[1:Wout + 1, :].set(Mw)
    mwt = jnp.stack([mw_pad[kw:kw + Wout, :].T for kw in range(3)])
    w9t = w.reshape(9, C)
    body = functools.partial(_upconv_body, TH=TH, Hin=Hin, Win=Win, C=C)
    return pl.pallas_call(
        body,
        out_shape=jax.ShapeDtypeStruct((B, 1, Hout, Wout), f32),
        grid_spec=pltpu.PrefetchScalarGridSpec(
            num_scalar_prefetch=0,
            grid=(B, n_tiles),
            in_specs=[
                pl.BlockSpec((1, Hin, C, Win), lambda b, i: (b, 0, 0, 0)),
                pl.BlockSpec((Hout + 8, Hin), lambda b, i: (0, 0)),
                pl.BlockSpec((3, Win, Wout), lambda b, i: (0, 0, 0)),
                pl.BlockSpec((9, C), lambda b, i: (0, 0)),
                pl.BlockSpec((1, 1), lambda b, i: (0, 0)),
                pl.BlockSpec((1, 1), lambda b, i: (0, 0)),
            ],
            out_specs=pl.BlockSpec((1, 1, TH, Wout),
                                   lambda b, i: (b, 0, i, 0)),
            scratch_shapes=[pltpu.VMEM((TH + 2, 9, Win), f32)]),
        compiler_params=pltpu.CompilerParams(
            dimension_semantics=("parallel", "parallel")),
    )(x_hcw, mh_pad, mwt, w9t, scale.reshape(1, 1), shift.reshape(1, 1))


def kernel(x_nchw, text, t_w, t_b, e1_w, e1_scale, e1_shift, e2_wx, e2_wt,
           e2_scale, e2_shift, e3_wx, e3_wt, e3_scale, e3_shift, d1_w,
           d1_scale, d1_shift, d2_wx, d2_wt, d2_scale, d2_shift, d3a_wx,
           d3a_wt, d3a_scale, d3a_shift, d3b_w, d3b_scale, d3b_shift):
    B = x_nchw.shape[0]
    x = jnp.transpose(x_nchw, (0, 2, 3, 1)).astype(f32)
    x = jnp.pad(x, ((0, 0), (0, 0), (0, 0), (0, 5)))
    text = text.astype(f32)

    layers = [(e2_wt, e2_scale, 0), (e3_wt, e3_scale, 64),
              (d2_wt, d2_scale, 128), (d3a_wt, d3a_scale, 192)]
    segs, bsegs = [], []
    for wt_, sc_, off in layers:
        ws = _wstar9(wt_) * sc_
        segs.append(jnp.einsum('kc,tco->kto', t_w[:, off:off + 64],
                               ws).reshape(512, -1))
        bsegs.append(jnp.einsum('c,tco->to', t_b[off:off + 64],
                                ws).reshape(-1))
    tball = _linear(text, jnp.concatenate(segs, axis=1),
                    jnp.concatenate(bsegs))
    tbs, o = [], 0
    for wt_, _, _ in layers:
        n = 9 * wt_.shape[-1]
        tbs.append(tball[:, o:o + n].reshape(B, 9, wt_.shape[-1]))
        o += n
    tb_e2, tb_e3, tb_d2, tb_d3a = tbs

    def r9(w):
        return w.reshape(9, w.shape[2], w.shape[3])

    e1w = jnp.pad(e1_w, ((0, 0), (0, 0), (0, 5), (0, 0)))
    h = _conv(x, r9(e1w), e1_scale, e1_shift, act='relu')
    h = _conv(h, r9(e2_wx), e2_scale, e2_shift, act='relu', tb=tb_e2)
    h = _conv(h, r9(e3_wx), e3_scale, e3_shift, act='relu', tb=tb_e3,
              pool=True)
    h = _conv(h, r9(d1_w), d1_scale, d1_shift, act='relu')
    h = _conv(h, r9(d2_wx), d2_scale, d2_shift, act='relu', tb=tb_d2)
    h = _conv(h, r9(d3a_wx), d3a_scale, d3a_shift, act='relu', tb=tb_d3a,
              hcw=True)
    return _upconv(h, d3b_w, d3b_scale, d3b_shift)


# speedup vs baseline: 1.7068x; 1.1293x over previous
import functools

import jax
import jax.numpy as jnp
from jax.experimental import pallas as pl
from jax.experimental.pallas import tpu as pltpu

f32 = jnp.float32


def _conv_body(*refs, TH, W, Wp, Cin, Cout, H, act, n_tiles, pool, has_tb,
               hcw=False):
    if has_tb:
        xm, xtop, xbot, w_ref, sc_ref, sh_ref, tb_ref, o_ref, xbuf = refs
    else:
        xm, xtop, xbot, w_ref, sc_ref, sh_ref, o_ref, xbuf = refs
        tb_ref = None
    i = pl.program_id(1)
    nf = (i > 0).astype(f32)
    nl = (i < n_tiles - 1).astype(f32)
    dt = xbuf.dtype
    xbuf[:, W:Wp, :] = jnp.zeros((TH + 4, Wp - W, Cin), dt)
    xbuf[0:1, 0:W, :] = jnp.zeros((1, W, Cin), dt)
    xbuf[TH + 3:TH + 4, 0:W, :] = jnp.zeros((1, W, Cin), dt)
    xbuf[2:TH + 2, 0:W, :] = xm[0]
    xbuf[1:2, 0:W, :] = (xtop[0] * nf).astype(dt)
    xbuf[TH + 2:TH + 3, 0:W, :] = (xbot[0] * nl).astype(dt)

    flat = xbuf[...].reshape((TH + 4) * Wp, Cin)
    acc = jnp.zeros((TH * Wp, Cout), f32)
    for dw in (-1, 0, 1):
        xs = flat[Wp + dw: Wp + dw + (TH + 2) * Wp, :]
        for kh in range(3):
            acc = acc + jnp.dot(xs[kh * Wp:(kh + TH) * Wp, :],
                                w_ref[kh * 3 + dw + 1],
                                preferred_element_type=f32)
    acc = acc * sc_ref[...] + sh_ref[...]
    a3 = acc.reshape(TH, Wp, Cout)

    if tb_ref is not None:
        tb = tb_ref[0]

        def v(k):
            return tb[k:k + 1, :].reshape(1, 1, Cout)

        rr = jax.lax.broadcasted_iota(jnp.int32, (TH, 1, 1), 0) + i * TH
        mt = (rr == 0).astype(f32)
        mb = (rr == H - 1).astype(f32)
        cc = jax.lax.broadcasted_iota(jnp.int32, (1, Wp, 1), 1)
        ml = (cc == 0).astype(f32)
        mr = (cc == W - 1).astype(f32)
        a3 = a3 + (v(0) + mt * v(1) + mb * v(2))
        a3 = a3 + ml * (v(3) + mt * v(5) + mb * v(7))
        a3 = a3 + mr * (v(4) + mt * v(6) + mb * v(8))

    if act == "relu":
        a3 = jnp.maximum(a3, 0.0)
    elif act == "sigmoid":
        a3 = 1.0 / (1.0 + jnp.exp(-a3))

    if pool:
        a = a3[:, 0:W, :].reshape(TH // 2, 2, W, Cout).max(axis=1)
        a = a.reshape(TH // 2, W // 2, 2, Cout).max(axis=2)
        o_ref[0] = a.astype(o_ref.dtype)
    elif hcw:
        o_ref[0] = jnp.swapaxes(a3[:, 0:W, :], 1, 2).astype(o_ref.dtype)
    else:
        o_ref[0] = a3[:, 0:W, :].astype(o_ref.dtype)


def _conv(x, w9, scale, shift, *, act, tb=None, pool=False, hcw=False, TH=32):
    B, H, W, Cin = x.shape
    Cout = w9.shape[-1]
    TH = min(TH, H)
    n_tiles = H // TH
    Wp = W + 8
    body = functools.partial(
        _conv_body, TH=TH, W=W, Wp=Wp, Cin=Cin, Cout=Cout, H=H, act=act,
        n_tiles=n_tiles, pool=pool, has_tb=tb is not None, hcw=hcw)
    in_specs = [
        pl.BlockSpec((1, TH, W, Cin), lambda b, i: (b, i, 0, 0)),
        pl.BlockSpec((1, 1, W, Cin),
                     lambda b, i: (b, jnp.maximum(i * TH - 1, 0), 0, 0)),
        pl.BlockSpec((1, 1, W, Cin),
                     lambda b, i: (b, jnp.minimum(i * TH + TH, H - 1), 0, 0)),
        pl.BlockSpec((9, Cin, Cout), lambda b, i: (0, 0, 0)),
        pl.BlockSpec((1, Cout), lambda b, i: (0, 0)),
        pl.BlockSpec((1, Cout), lambda b, i: (0, 0)),
    ]
    inputs = [x, x, x, w9, scale.reshape(1, Cout), shift.reshape(1, Cout)]
    if tb is not None:
        in_specs.append(pl.BlockSpec((1, 9, Cout), lambda b, i: (b, 0, 0)))
        inputs.append(tb)
    out_dt = x.dtype
    if pool:
        out_shape = jax.ShapeDtypeStruct((B, H // 2, W // 2, Cout), out_dt)
        out_spec = pl.BlockSpec((1, TH // 2, W // 2, Cout),
                                lambda b, i: (b, i, 0, 0))
    elif hcw:
        out_shape = jax.ShapeDtypeStruct((B, H, Cout, W), out_dt)
        out_spec = pl.BlockSpec((1, TH, Cout, W), lambda b, i: (b, i, 0, 0))
    else:
        out_shape = jax.ShapeDtypeStruct((B, H, W, Cout), out_dt)
        out_spec = pl.BlockSpec((1, TH, W, Cout), lambda b, i: (b, i, 0, 0))
    return pl.pallas_call(
        body,
        out_shape=out_shape,
        grid_spec=pltpu.PrefetchScalarGridSpec(
            num_scalar_prefetch=0,
            grid=(B, n_tiles),
            in_specs=in_specs,
            out_specs=out_spec,
            scratch_shapes=[pltpu.VMEM((TH + 4, Wp, Cin), x.dtype)]),
        compiler_params=pltpu.CompilerParams(
            dimension_semantics=("parallel", "parallel")),
    )(*inputs)


def _lin_body(x_ref, w_ref, b_ref, o_ref):
    o_ref[...] = (jnp.dot(x_ref[...], w_ref[...],
                          preferred_element_type=f32) + b_ref[...])


def _linear(x, w, b):
    B, K = x.shape
    N = w.shape[-1]
    return pl.pallas_call(
        _lin_body,
        out_shape=jax.ShapeDtypeStruct((B, N), f32),
        grid=(1,),
        in_specs=[pl.BlockSpec((B, K), lambda i: (0, 0)),
                  pl.BlockSpec((K, N), lambda i: (0, 0)),
                  pl.BlockSpec((1, N), lambda i: (0, 0))],
        out_specs=pl.BlockSpec((B, N), lambda i: (0, 0)),
    )(x, w, b.reshape(1, N))


def _wstar9(wt):
    full = wt.sum((0, 1))
    top = -wt[0].sum(0)
    bot = -wt[2].sum(0)
    left = -wt[:, 0].sum(0)
    right = -wt[:, 2].sum(0)
    return jnp.stack([full, top, bot, left, right,
                      wt[0, 0], wt[0, 2], wt[2, 0], wt[2, 2]])


def _bilin_mat(n_in, n_out):
    i = jnp.arange(n_out, dtype=f32)
    src = i * (n_in - 1) / (n_out - 1)
    i0 = jnp.clip(jnp.floor(src).astype(jnp.int32), 0, n_in - 2)
    frac = src - i0.astype(f32)
    rows = jnp.arange(n_out)
    M = jnp.zeros((n_out, n_in), f32)
    M = M.at[rows, i0].add(1.0 - frac)
    M = M.at[rows, i0 + 1].add(frac)
    return M


def _upconv_body(x_ref, mh_ref, mwt_ref, w9t_ref, sc_ref, sh_ref, o_ref,
                 zbuf, *, TH, Hin, Win, C):
    i = pl.program_id(1)
    xflat = x_ref[0].reshape(Hin, C * Win)
    mh = mh_ref[pl.ds(i * TH, TH + 2), :]
    uph = jnp.dot(mh, xflat, preferred_element_type=f32)
    uph3 = uph.reshape(TH + 2, C, Win)
    w9t = w9t_ref[...]
    for r in range(TH + 2):
        zbuf[r] = jnp.dot(w9t, uph3[r], preferred_element_type=f32)
    zb = zbuf[...]
    acc = jnp.zeros((TH, 2 * Win), f32)
    for kh in range(3):
        for kw in range(3):
            zs = zb[kh:kh + TH, 3 * kh + kw, :]
            acc = acc + jnp.dot(zs, mwt_ref[kw],
                                preferred_element_type=f32)
    acc = acc * sc_ref[0, 0] + sh_ref[0, 0]
    o_ref[0, 0] = 1.0 / (1.0 + jnp.exp(-acc))


def _upconv(x_hcw, w, scale, shift, *, TH=32):
    B, Hin, C, Win = x_hcw.shape
    Hout, Wout = 2 * Hin, 2 * Win
    TH = min(TH, Hout)
    n_tiles = Hout // TH
    Mh = _bilin_mat(Hin, Hout)
    mh_pad = jnp.zeros((Hout + 8, Hin), f32).at[1:Hout + 1, :].set(Mh)
    Mw = _bilin_mat(Win, Wout)
    mw_pad = jnp.zeros((Wout + 2, Win), f32).at[1:Wout + 1, :].set(Mw)
    mwt = jnp.stack([mw_pad[kw:kw + Wout, :].T for kw in range(3)])
    w9t = w.reshape(9, C)
    body = functools.partial(_upconv_body, TH=TH, Hin=Hin, Win=Win, C=C)
    return pl.pallas_call(
        body,
        out_shape=jax.ShapeDtypeStruct((B, 1, Hout, Wout), f32),
        grid_spec=pltpu.PrefetchScalarGridSpec(
            num_scalar_prefetch=0,
            grid=(B, n_tiles),
            in_specs=[
                pl.BlockSpec((1, Hin, C, Win), lambda b, i: (b, 0, 0, 0)),
                pl.BlockSpec((Hout + 8, Hin), lambda b, i: (0, 0)),
                pl.BlockSpec((3, Win, Wout), lambda b, i: (0, 0, 0)),
                pl.BlockSpec((9, C), lambda b, i: (0, 0)),
                pl.BlockSpec((1, 1), lambda b, i: (0, 0)),
                pl.BlockSpec((1, 1), lambda b, i: (0, 0)),
            ],
            out_specs=pl.BlockSpec((1, 1, TH, Wout),
                                   lambda b, i: (b, 0, i, 0)),
            scratch_shapes=[pltpu.VMEM((TH + 2, 9, Win), f32)]),
        compiler_params=pltpu.CompilerParams(
            dimension_semantics=("parallel", "parallel")),
    )(x_hcw, mh_pad, mwt, w9t, scale.reshape(1, 1), shift.reshape(1, 1))


def kernel(x_nchw, text, t_w, t_b, e1_w, e1_scale, e1_shift, e2_wx, e2_wt,
           e2_scale, e2_shift, e3_wx, e3_wt, e3_scale, e3_shift, d1_w,
           d1_scale, d1_shift, d2_wx, d2_wt, d2_scale, d2_shift, d3a_wx,
           d3a_wt, d3a_scale, d3a_shift, d3b_w, d3b_scale, d3b_shift):
    B = x_nchw.shape[0]
    x = jnp.transpose(x_nchw, (0, 2, 3, 1)).astype(jnp.bfloat16)
    x = jnp.pad(x, ((0, 0), (0, 0), (0, 0), (0, 5)))
    text = text.astype(f32)

    layers = [(e2_wt, e2_scale, 0), (e3_wt, e3_scale, 64),
              (d2_wt, d2_scale, 128), (d3a_wt, d3a_scale, 192)]
    segs, bsegs = [], []
    for wt_, sc_, off in layers:
        ws = _wstar9(wt_) * sc_
        segs.append(jnp.einsum('kc,tco->kto', t_w[:, off:off + 64],
                               ws).reshape(512, -1))
        bsegs.append(jnp.einsum('c,tco->to', t_b[off:off + 64],
                                ws).reshape(-1))
    tball = _linear(text, jnp.concatenate(segs, axis=1),
                    jnp.concatenate(bsegs))
    tbs, o = [], 0
    for wt_, _, _ in layers:
        n = 9 * wt_.shape[-1]
        tbs.append(tball[:, o:o + n].reshape(B, 9, wt_.shape[-1]))
        o += n
    tb_e2, tb_e3, tb_d2, tb_d3a = tbs

    def r9(w):
        return w.reshape(9, w.shape[2], w.shape[3])

    e1w = jnp.pad(e1_w, ((0, 0), (0, 0), (0, 5), (0, 0)))
    h = _conv(x, r9(e1w), e1_scale, e1_shift, act='relu')
    h = _conv(h, r9(e2_wx), e2_scale, e2_shift, act='relu', tb=tb_e2)
    h = _conv(h, r9(e3_wx), e3_scale, e3_shift, act='relu', tb=tb_e3,
              pool=True)
    h = _conv(h, r9(d1_w), d1_scale, d1_shift, act='relu')
    h = _conv(h, r9(d2_wx), d2_scale, d2_shift, act='relu', tb=tb_d2)
    h = _conv(h, r9(d3a_wx), d3a_scale, d3a_shift, act='relu', tb=tb_d3a,
              hcw=True)
    return _upconv(h, d3b_w, d3b_scale, d3b_shift)
